# Initial kernel scaffold; baseline (speedup 1.0000x reference)
import jax
import jax.numpy as jnp
from jax.experimental import pallas as pl


def kernel(x, edge_index, batch, params):
    raise NotImplementedError("write your pallas kernel here")


# speedup vs baseline: 17.5730x; 17.5730x over previous
import jax
import jax.numpy as jnp
from jax import lax
from jax.experimental import pallas as pl
from jax.experimental.pallas import tpu as pltpu
from jax.experimental.pallas import tpu_sc as plsc

_N = 10000
_E = 320000
_G = 64
_INC = 128
_H = 64
_EPS = 1e-5

_NC = 2
_NS = 16
_NW = _NC * _NS
_EW = _E // _NW
_CH = 128
_NCH = -(-_EW // _CH) + (-(-_EW // _CH)) % 2
_EWP = _NCH * _CH
_RPT = 632
_NPAD = _RPT * _NS

_HI = lax.Precision.DEFAULT
_C10 = (((1,), (0,)), ((), ()))
_C00 = (((0,), (0,)), ((), ()))

_mesh = plsc.VectorSubcoreMesh(core_axis_name="c", subcore_axis_name="s")
_TCP = pltpu.CompilerParams(vmem_limit_bytes=100 * 2**20)

_NSP = 2
_WS = _H // _NSP


def _msg_body(*refs):
    hws = refs[0:_NSP]
    srcp, dstp, zblk = refs[_NSP:_NSP + 3]
    outs = refs[_NSP + 3:2 * _NSP + 3]
    k = 2 * _NSP + 3
    src_v, dst_v = refs[k:k + 2]
    bufa = refs[k + 2:k + 2 + _NSP]
    bufb = refs[k + 2 + _NSP:k + 2 + 2 * _NSP]
    accs = refs[k + 2 + 2 * _NSP:k + 2 + 3 * _NSP]
    sema, semb, semsc = refs[k + 2 + 3 * _NSP:]

    c = lax.axis_index("c")
    s = lax.axis_index("s")
    w = c * _NS + s
    pltpu.sync_copy(srcp.at[w], src_v)
    pltpu.sync_copy(dstp.at[w], dst_v)
    rows = pl.ds(s * _RPT, _RPT)
    for a in accs:
        pltpu.sync_copy(zblk, a.at[rows])
    plsc.subcore_barrier()

    def gather(j, bufs, sem):
        idx = src_v.at[pl.ds(j * _CH, _CH)]
        for hwp, b in zip(hws, bufs):
            pltpu.async_copy(hwp.at[idx], b, sem)

    def gwait(bufs, sem):
        idx = src_v.at[pl.ds(0, _CH)]
        for hwp, b in zip(hws, bufs):
            pltpu.make_async_copy(hwp.at[idx], b, sem).wait()

    def scat(j, bufs):
        idx = dst_v.at[j]
        pltpu.async_copy(bufs[0], accs[0].at[idx], semsc, add=True)
        pltpu.sync_copy(bufs[1], accs[1].at[idx], add=True)
        pltpu.make_async_copy(bufs[0], accs[0].at[idx], semsc).wait()

    gather(0, bufa, sema)
    gather(1, bufb, semb)

    def body(i, _):
        j = i * 2
        gwait(bufa, sema)
        scat(j, bufa)

        @pl.when(j + 2 < _NCH)
        def _():
            gather(j + 2, bufa, sema)
        gwait(bufb, semb)
        scat(j + 1, bufb)

        @pl.when(j + 3 < _NCH)
        def _():
            gather(j + 3, bufb, semb)
        return 0
    lax.fori_loop(0, _NCH // 2, body, 0)
    plsc.subcore_barrier()
    for a, o in zip(accs, outs):
        pltpu.sync_copy(a.at[rows], o.at[c, rows])


_msg = pl.kernel(
    _msg_body,
    out_type=[jax.ShapeDtypeStruct((_NC, _NPAD, _WS), jnp.float32)] * _NSP,
    mesh=_mesh,
    scratch_types=(
        [pltpu.VMEM((_EWP,), jnp.int32),
         pltpu.VMEM((_NCH, _CH), jnp.int32)]
        + [pltpu.VMEM((_CH, _WS), jnp.float32)] * (2 * _NSP)
        + [pltpu.VMEM_SHARED((_NPAD, _WS), jnp.float32)] * _NSP
        + [pltpu.SemaphoreType.DMA] * 3
    ),
    compiler_params=pltpu.CompilerParams(use_tc_tiling_on_sc=False),
)


def _degk_body(ones_hbm, dstp, zblk, out, dst_v, buf, acc):
    c = lax.axis_index("c")
    s = lax.axis_index("s")
    w = c * _NS + s
    pltpu.sync_copy(dstp.at[w], dst_v)
    pltpu.sync_copy(ones_hbm, buf)
    pltpu.sync_copy(zblk, acc.at[pl.ds(s * _RPT, _RPT)])
    plsc.subcore_barrier()

    def body(j, _):
        pltpu.sync_copy(buf, acc.at[dst_v.at[j]], add=True)
        return 0
    lax.fori_loop(0, _NCH, body, 0)
    plsc.subcore_barrier()
    pltpu.sync_copy(acc.at[pl.ds(s * _RPT, _RPT)],
                    out.at[c, pl.ds(s * _RPT, _RPT)])


_degk = pl.kernel(
    _degk_body,
    out_type=jax.ShapeDtypeStruct((_NC, _NPAD, 16), jnp.float32),
    mesh=_mesh,
    scratch_types=[
        pltpu.VMEM((_NCH, _CH), jnp.int32),
        pltpu.VMEM((_CH, 16), jnp.float32),
        pltpu.VMEM_SHARED((_NPAD, 16), jnp.float32),
    ],
    compiler_params=pltpu.CompilerParams(use_tc_tiling_on_sc=False),
)


def _bn_relu(accl, acch, hwl, hwh, dis, b, g, t):
    accs = jnp.concatenate(
        [accl[0, 0:_N, :] + accl[1, 0:_N, :],
         acch[0, 0:_N, :] + acch[1, 0:_N, :]], axis=1)
    hw = jnp.concatenate([hwl[...], hwh[...]], axis=1)
    sx = dis[...] * (accs + hw) + b[...]
    mu = jnp.mean(sx, axis=0, keepdims=True)
    xc = sx - mu
    var = jnp.mean(xc * xc, axis=0, keepdims=True)
    return jnp.maximum(g[...] * xc * lax.rsqrt(var + _EPS) + t[...], 0.0)


def _split_hw(hwl_o, hwh_o, dis, h, wn):
    hw = dis[...] * lax.dot_general(h, wn[...], _C10, precision=_HI,
                                    preferred_element_type=jnp.float32)
    hwl_o[...] = hw[:, 0:_WS]
    hwh_o[...] = hw[:, _WS:_H]


def _prep_body(x, w0, degp, dis_o, hwl_o, hwh_o):
    deg = degp[0, 0:_N, 0:1] + degp[1, 0:_N, 0:1] + 1.0
    dis = lax.rsqrt(deg)
    dis_o[...] = dis
    _split_hw(hwl_o, hwh_o, dis_o, x[...], w0)


_prep = pl.pallas_call(
    _prep_body,
    out_shape=[jax.ShapeDtypeStruct((_N, 1), jnp.float32),
               jax.ShapeDtypeStruct((_N, _WS), jnp.float32),
               jax.ShapeDtypeStruct((_N, _WS), jnp.float32)],
    compiler_params=_TCP,
)


def _post0_body(accl, acch, hwl, hwh, dis, b, g, t, wn, h_o, hwl_o, hwh_o):
    h = _bn_relu(accl, acch, hwl, hwh, dis, b, g, t)
    h_o[...] = h
    _split_hw(hwl_o, hwh_o, dis, h, wn)


_post0 = pl.pallas_call(
    _post0_body,
    out_shape=[jax.ShapeDtypeStruct((_N, _H), jnp.float32),
               jax.ShapeDtypeStruct((_N, _WS), jnp.float32),
               jax.ShapeDtypeStruct((_N, _WS), jnp.float32)],
    compiler_params=_TCP,
)


def _mid_body(accl, acch, hwl, hwh, dis, b, g, t, res, wn, h_o, hwl_o, hwh_o):
    h = res[...] + _bn_relu(accl, acch, hwl, hwh, dis, b, g, t)
    h_o[...] = h
    _split_hw(hwl_o, hwh_o, dis, h, wn)


_mid = pl.pallas_call(
    _mid_body,
    out_shape=[jax.ShapeDtypeStruct((_N, _H), jnp.float32),
               jax.ShapeDtypeStruct((_N, _WS), jnp.float32),
               jax.ShapeDtypeStruct((_N, _WS), jnp.float32)],
    compiler_params=_TCP,
)


def _fin_body(accl, acch, hwl, hwh, dis, b, g, t, res, bcol,
              wh, bh, wo, bo, out_o):
    y = res[...] + _bn_relu(accl, acch, hwl, hwh, dis, b, g, t)
    onehot = (bcol[...] == lax.broadcasted_iota(jnp.int32, (_N, _G), 1)
              ).astype(jnp.float32)
    sums = lax.dot_general(onehot, y, _C00, precision=_HI,
                           preferred_element_type=jnp.float32)
    cnt = lax.dot_general(onehot, jnp.ones((_N, 1), jnp.float32), _C00,
                          precision=_HI, preferred_element_type=jnp.float32)
    pooled = sums / jnp.maximum(cnt, 1.0)
    hm = jnp.maximum(lax.dot_general(pooled, wh[...], _C10, precision=_HI,
                                     preferred_element_type=jnp.float32)
                     + bh[...], 0.0)
    out_o[...] = lax.dot_general(hm, wo[...], _C10, precision=_HI,
                                 preferred_element_type=jnp.float32) + bo[...]


_fin = pl.pallas_call(
    _fin_body,
    out_shape=jax.ShapeDtypeStruct((_G, 1), jnp.float32),
    compiler_params=_TCP,
)


def kernel(x, edge_index, batch, params):
    p = params
    src = edge_index[0].reshape(_NW, _EW)
    dst = edge_index[1].reshape(_NW, _EW)
    pad = _EWP - _EW
    srcp = jnp.concatenate(
        [src, jnp.zeros((_NW, pad), edge_index.dtype)], axis=1)
    dstp = jnp.concatenate(
        [dst, jnp.full((_NW, pad), _N, edge_index.dtype)], axis=1)
    dstp3 = dstp.reshape(_NW, _NCH, _CH)
    zblk = jnp.zeros((_RPT, _WS), jnp.float32)
    zblk16 = jnp.zeros((_RPT, 16), jnp.float32)
    ones16 = jnp.ones((_CH, 16), jnp.float32)
    bcol = batch.reshape(_N, 1)

    def v(k):
        return p[k].reshape(1, -1)

    degp = _degk(ones16, dstp3, zblk16)
    dis, hwl, hwh = _prep(x, p['W0'], degp)

    accl, acch = _msg(hwl, hwh, srcp, dstp3, zblk)
    h, hwl, hwh = _post0(accl, acch, hwl, hwh, dis,
                         v('b0'), v('g0'), v('t0'), p['W1'])
    for i in (1, 2):
        accl, acch = _msg(hwl, hwh, srcp, dstp3, zblk)
        h, hwl, hwh = _mid(accl, acch, hwl, hwh, dis,
                           v('b%d' % i), v('g%d' % i), v('t%d' % i),
                           h, p['W%d' % (i + 1)])
    accl, acch = _msg(hwl, hwh, srcp, dstp3, zblk)
    return _fin(accl, acch, hwl, hwh, dis, v('b3'), v('g3'), v('t3'), h, bcol,
                p['Wh'], v('bh'), p['Wo'], v('bo'))
